# baseline (speedup 1.0000x reference)
import functools

import jax
import jax.numpy as jnp
from jax import lax
from jax.experimental import pallas as pl
from jax.experimental.pallas import tpu as pltpu
from jax.experimental.pallas import tpu_sc as plsc

N_NODES = 10000
N_EDGES = 320000
DIM = 128
HD = DIM // 2

NC = 2
NS = 16
EPT = N_EDGES // NS
CH = 125
NCHUNK = EPT // CH
NBUF = 4
N_PAD = 10240
RPT = N_PAD // NS
ZCH = 120
LANES = 16


def _sc_aggregate(xs, src3, dst3):
    mesh = plsc.VectorSubcoreMesh(core_axis_name="c", subcore_axis_name="s")

    @functools.partial(
        pl.kernel,
        mesh=mesh,
        out_type=jax.ShapeDtypeStruct((N_PAD, DIM), jnp.float32),
        compiler_params=pltpu.CompilerParams(use_tc_tiling_on_sc=False),
        scratch_types=[
            pltpu.VMEM((NCHUNK, CH), jnp.int32),
            pltpu.VMEM((NCHUNK, CH), jnp.int32),
            pltpu.VMEM((NBUF, CH, HD), jnp.float32),
            pltpu.VMEM_SHARED((N_PAD, HD), jnp.float32),
            pltpu.SemaphoreType.DMA,
            pltpu.SemaphoreType.DMA,
        ],
    )
    def k(x_hbm, src_hbm, dst_hbm, out_hbm, sidx, didx, rows, acc, gsem, ssem):
        c = lax.axis_index("c")
        s = lax.axis_index("s")

        def zbody(t, _):
            i = t // (HD // LANES)
            j = t % (HD // LANES)
            rows[0, i, pl.ds(j * LANES, LANES)] = jnp.zeros((LANES,),
                                                            jnp.float32)
            return 0
        lax.fori_loop(0, ZCH * (HD // LANES), zbody, 0)
        for j in range(RPT // ZCH + 1):
            rr = min(ZCH, RPT - j * ZCH)
            pltpu.sync_copy(rows.at[0, pl.ds(0, rr)],
                            acc.at[pl.ds(s * RPT + j * ZCH, rr)])

        pltpu.sync_copy(src_hbm.at[s], sidx)
        pltpu.sync_copy(dst_hbm.at[s], didx)
        plsc.subcore_barrier()

        xc = x_hbm.at[c]
        for p in range(NBUF - 1):
            pltpu.async_copy(xc.at[sidx.at[p]], rows.at[p], gsem)

        def chunk_body(i, _):
            b = lax.rem(i, NBUF)
            pltpu.make_async_copy(
                xc.at[sidx.at[i]], rows.at[b], gsem).wait()
            pltpu.async_copy(rows.at[b], acc.at[didx.at[i]], ssem, add=True)

            nxt = i + NBUF - 1
            nb = lax.rem(nxt, NBUF)

            @pl.when(nxt < NCHUNK)
            def _prefetch():
                @pl.when(i >= 1)
                def _drain_one():
                    pltpu.make_async_copy(
                        rows.at[nb], acc.at[didx.at[i]], ssem).wait()
                pltpu.async_copy(xc.at[sidx.at[nxt]], rows.at[nb], gsem)
            return 0
        lax.fori_loop(0, NCHUNK, chunk_body, 0)

        for p in range(NBUF):
            pltpu.make_async_copy(
                rows.at[p], acc.at[didx.at[0]], ssem).wait()

        plsc.subcore_barrier()
        pltpu.sync_copy(acc.at[pl.ds(s * RPT, RPT)],
                        out_hbm.at[pl.ds(s * RPT, RPT), pl.ds(c * HD, HD)])

    return k(xs, src3, dst3)


def _tc_combine(x, agg, W1t, W2t):
    BR = 1000
    grid = N_NODES // BR

    def body(x_ref, a_ref, w1_ref, w2_ref, o_ref):
        acc = jnp.dot(x_ref[...], w1_ref[...],
                      preferred_element_type=jnp.float32)
        acc += jnp.dot(a_ref[...], w2_ref[...],
                       preferred_element_type=jnp.float32)
        o_ref[...] = jnp.maximum(acc, 0.0)

    return pl.pallas_call(
        body,
        grid=(grid,),
        in_specs=[
            pl.BlockSpec((BR, DIM), lambda i: (i, 0)),
            pl.BlockSpec((BR, DIM), lambda i: (i, 0)),
            pl.BlockSpec((DIM, DIM), lambda i: (0, 0)),
            pl.BlockSpec((DIM, DIM), lambda i: (0, 0)),
        ],
        out_specs=pl.BlockSpec((BR, DIM), lambda i: (i, 0)),
        out_shape=jax.ShapeDtypeStruct((N_NODES, DIM), jnp.float32),
    )(x, agg, W1t, W2t)


def kernel(x, edge_index, W1, W2):
    src = edge_index[0].astype(jnp.int32)
    dst = edge_index[1].astype(jnp.int32)
    src3 = src.reshape(NS, NCHUNK, CH)
    dst3 = dst.reshape(NS, NCHUNK, CH)
    xs = x.reshape(N_NODES, NC, HD).transpose(1, 0, 2)
    agg = _sc_aggregate(xs, src3, dst3)[:N_NODES]
    return _tc_combine(x, agg, W1.T, W2.T)


# speedup vs baseline: 1.0983x; 1.0983x over previous
import functools

import jax
import jax.numpy as jnp
from jax import lax
from jax.experimental import pallas as pl
from jax.experimental.pallas import tpu as pltpu
from jax.experimental.pallas import tpu_sc as plsc

N_NODES = 10000
N_EDGES = 320000
DIM = 128
HD = DIM // 2

NC = 2
NS = 16
EPT = N_EDGES // NS
CH = 80
NCHUNK = EPT // CH
NBUF = 4
N_PAD = 10240
RPT = N_PAD // NS
ZCH = 120
LANES = 16


def _sc_aggregate(x2, edges, dst3):
    mesh = plsc.VectorSubcoreMesh(core_axis_name="c", subcore_axis_name="s")

    @functools.partial(
        pl.kernel,
        mesh=mesh,
        out_type=jax.ShapeDtypeStruct((N_PAD, DIM), jnp.float32),
        compiler_params=pltpu.CompilerParams(use_tc_tiling_on_sc=False),
        scratch_types=[
            pltpu.VMEM((EPT,), jnp.int32),
            pltpu.VMEM((NCHUNK, CH), jnp.int32),
            pltpu.VMEM((NBUF, CH, HD), jnp.float32),
            pltpu.VMEM_SHARED((N_PAD, HD), jnp.float32),
            pltpu.SemaphoreType.DMA,
            pltpu.SemaphoreType.DMA,
        ],
    )
    def k(x_hbm, e_hbm, dst_hbm, out_hbm, sidx, didx, rows, acc, gsem, ssem):
        c = lax.axis_index("c")
        s = lax.axis_index("s")

        pltpu.sync_copy(e_hbm.at[0, pl.ds(s * EPT, EPT)], sidx)
        pltpu.sync_copy(dst_hbm.at[s], didx)

        def tbody(t, _):
            sl = pl.ds(t * LANES, LANES)
            sidx[sl] = 2 * sidx[sl] + c
            return 0

        head = ((NBUF - 1) * CH + LANES - 1) // LANES
        lax.fori_loop(0, head, tbody, 0)
        for p in range(NBUF - 1):
            pltpu.async_copy(
                x_hbm.at[sidx.at[pl.ds(p * CH, CH)]], rows.at[p], gsem)
        lax.fori_loop(head, EPT // LANES, tbody, 0)

        def zbody(t, _):
            i = t // (HD // LANES)
            j = t % (HD // LANES)
            rows[NBUF - 1, i, pl.ds(j * LANES, LANES)] = jnp.zeros(
                (LANES,), jnp.float32)
            return 0
        lax.fori_loop(0, ZCH * (HD // LANES), zbody, 0)
        for j in range(RPT // ZCH + 1):
            rr = min(ZCH, RPT - j * ZCH)
            pltpu.sync_copy(rows.at[NBUF - 1, pl.ds(0, rr)],
                            acc.at[pl.ds(s * RPT + j * ZCH, rr)])
        plsc.subcore_barrier()

        def chunk_body(i, _):
            b = lax.rem(i, NBUF)
            pltpu.make_async_copy(
                x_hbm.at[sidx.at[pl.ds(i * CH, CH)]], rows.at[b], gsem).wait()
            pltpu.async_copy(rows.at[b], acc.at[didx.at[i]], ssem, add=True)

            nxt = i + NBUF - 1
            nb = lax.rem(nxt, NBUF)

            @pl.when(nxt < NCHUNK)
            def _prefetch():
                @pl.when(i >= 1)
                def _drain_one():
                    pltpu.make_async_copy(
                        rows.at[nb], acc.at[didx.at[i]], ssem).wait()
                pltpu.async_copy(
                    x_hbm.at[sidx.at[pl.ds(nxt * CH, CH)]], rows.at[nb], gsem)
            return 0
        lax.fori_loop(0, NCHUNK, chunk_body, 0)

        for p in range(NBUF):
            pltpu.make_async_copy(
                rows.at[p], acc.at[didx.at[0]], ssem).wait()

        plsc.subcore_barrier()
        pltpu.sync_copy(acc.at[pl.ds(s * RPT, RPT)],
                        out_hbm.at[pl.ds(s * RPT, RPT), pl.ds(c * HD, HD)])

    return k(x2, edges, dst3)


def _tc_combine(x, agg, W1t, W2t):
    BR = 1000
    grid = N_NODES // BR

    def body(x_ref, a_ref, w1_ref, w2_ref, o_ref):
        acc = jnp.dot(x_ref[...], w1_ref[...],
                      preferred_element_type=jnp.float32)
        acc += jnp.dot(a_ref[...], w2_ref[...],
                       preferred_element_type=jnp.float32)
        o_ref[...] = jnp.maximum(acc, 0.0)

    return pl.pallas_call(
        body,
        grid=(grid,),
        in_specs=[
            pl.BlockSpec((BR, DIM), lambda i: (i, 0)),
            pl.BlockSpec((BR, DIM), lambda i: (i, 0)),
            pl.BlockSpec((DIM, DIM), lambda i: (0, 0)),
            pl.BlockSpec((DIM, DIM), lambda i: (0, 0)),
        ],
        out_specs=pl.BlockSpec((BR, DIM), lambda i: (i, 0)),
        out_shape=jax.ShapeDtypeStruct((N_NODES, DIM), jnp.float32),
    )(x, agg, W1t, W2t)


def kernel(x, edge_index, W1, W2):
    edges = edge_index.astype(jnp.int32)
    dst3 = edges[1].reshape(NS, NCHUNK, CH)
    x2 = x.reshape(2 * N_NODES, HD)
    agg = _sc_aggregate(x2, edges, dst3)
    return _tc_combine(x, agg, W1.T, W2.T)
